# baseline (speedup 1.0000x reference)
import jax
import jax.numpy as jnp
from jax.experimental import pallas as pl
from jax.experimental.pallas import tpu as pltpu

_C = 64


def _softplus(z):
    return jnp.maximum(z, 0.0) + jnp.log1p(jnp.exp(-jnp.abs(z)))


def _sigmoid(z):
    return 1.0 / (1.0 + jnp.exp(-z))


def _block_kernel(xp_ref, xn_ref, wwT_ref, wbgT_ref, waT_ref, woT_ref,
                  dtb_ref, nega_ref, out_ref, S_ref, prev_ref, v2_ref,
                  wkcd_ref, rkdec_ref, wkdw_ref, attn_ref, gate_ref, sdec_ref):
    C = _C
    H, d, _ = S_ref.shape
    D = H * d
    f32 = jnp.float32
    k = pl.program_id(1)

    sls = [slice(h * d, (h + 1) * d) for h in range(H)]

    boths = []
    for h in range(H):
        lhs = jnp.concatenate([wkcd_ref[:, sls[h]], rkdec_ref[:, sls[h]]],
                              axis=0)
        boths.append(jnp.dot(lhs, S_ref[h], preferred_element_type=f32))

    xb = xn_ref[0]
    bg = jnp.dot(xb, wbgT_ref[...], preferred_element_type=f32)
    beta = _sigmoid(bg[:, 0:H])
    gate = _sigmoid(bg[:, H:2 * H])
    z = jnp.dot(xb, waT_ref[...], preferred_element_type=f32) + dtb_ref[...]
    decay = nega_ref[...] * _softplus(z)
    xb16 = xb.astype(jnp.bfloat16)
    v_full = jnp.dot(xb16, wwT_ref[...], preferred_element_type=f32)

    vns, o_parts = [], []
    for h in range(H):
        v_new = v2_ref[:, sls[h]] - boths[h][:C]
        vns.append(v_new)
        o_parts.append(boths[h][C:] + jnp.dot(
            attn_ref[h * C:(h + 1) * C, :], v_new, preferred_element_type=f32))

    ri = jax.lax.broadcasted_iota(jnp.int32, (C, C), 0)
    ci = jax.lax.broadcasted_iota(jnp.int32, (C, C), 1)
    eyeC = jnp.where(ri == ci, f32(1.0), f32(0.0))
    L1 = jnp.where(ri >= ci, f32(1.0), f32(0.0))
    subD = jnp.where(ri == ci + 1, f32(1.0), f32(0.0))
    m0 = jax.lax.broadcasted_iota(jnp.int32, (C, D), 0) == 0
    G = min(8, H)
    riS = jax.lax.broadcasted_iota(jnp.int32, (C, G * C), 0)
    ciS = jax.lax.broadcasted_iota(jnp.int32, (C, G * C), 1)
    eyeS = jnp.where((ciS & (C - 1)) == riS, f32(1.0), f32(0.0))
    rB = jax.lax.broadcasted_iota(jnp.int32, (G * C, G * C), 0)
    cB = jax.lax.broadcasted_iota(jnp.int32, (G * C, G * C), 1)
    blk = (rB >> 6) == (cB >> 6)

    dec = jnp.dot(L1, decay, preferred_element_type=f32)
    decT = jax.lax.dot_general(dec, eyeC, (((0,), (0,)), ((), ())),
                               preferred_element_type=f32)

    rks = []
    for h in range(H):
        xh = xb[:, sls[h]]
        inv = 1.0 / jnp.maximum(
            jnp.sqrt(jnp.sum(xh * xh, axis=1, keepdims=True)), f32(1e-12))
        rks.append(xh * inv)
    rk_all = jnp.concatenate(rks, axis=1)
    prev_row = jnp.where(k == 0, f32(0.0), prev_ref[...])
    wk_all = jnp.dot(subD, rk_all, preferred_element_type=f32) \
        + jnp.where(m0, jnp.broadcast_to(prev_row, (C, D)), f32(0.0))
    prev_ref[...] = rk_all[C - 1:C, :]

    S_news, outs = [], []
    for h in range(H):
        S_news.append(S_ref[h] * sdec_ref[:, h:h + 1] + jax.lax.dot_general(
            wkdw_ref[:, sls[h]], vns[h], (((0,), (0,)), ((), ())),
            preferred_element_type=f32))
        outs.append(o_parts[h] * gate_ref[:, h:h + 1])
    for h in range(H):
        S_ref[h] = S_news[h]
    o_full = jnp.concatenate(outs, axis=1).astype(jnp.bfloat16)
    out_ref[0] = xp_ref[0] + jnp.dot(o_full, woT_ref[...],
                                     preferred_element_type=f32)

    Ms, wkbs = [], []
    for h in range(H):
        wkb = wk_all[:, sls[h]] * beta[:, h:h + 1]
        rawb = jax.lax.dot_general(
            jnp.concatenate([wkb, rk_all[:, sls[h]]], axis=0), wk_all[:, sls[h]],
            (((1,), (1,)), ((), ())), preferred_element_type=f32)
        L = jnp.exp(jnp.where(ri >= ci, dec[:, h:h + 1] - decT[h:h + 1, :],
                              f32(-1e30)))
        Ms.append(jnp.where(ri > ci, rawb[:C] * L, f32(0.0)))
        attn_ref[h * C:(h + 1) * C, :] = rawb[C:] * L
        wkbs.append(wkb)

    def bd(Q):
        return jnp.where(blk, pltpu.repeat(Q, G, axis=0), f32(0.0))

    A_gs = []
    for g in range(H // G):
        P = -jnp.concatenate(Ms[g * G:(g + 1) * G], axis=1)
        A = eyeS + P
        Q = jnp.dot(P, bd(P), preferred_element_type=f32)
        for i in range(5):
            Qb = bd(Q)
            if i < 4:
                both2 = jnp.dot(jnp.concatenate([A, Q], axis=0), Qb,
                                preferred_element_type=f32)
                A = A + both2[:C]
                Q = both2[C:]
            else:
                A = A + jnp.dot(A, Qb, preferred_element_type=f32)
        A_gs.append(A)
    A = jnp.concatenate(A_gs, axis=1)

    sdec_ref[...] = jnp.exp(dec[C - 1:C, :])
    gate_ref[...] = gate
    for h in range(H):
        A_h = A[:, h * C:(h + 1) * C]
        dec_h = dec[:, h:h + 1]
        dec_exp = jnp.exp(dec_h)
        vh = v_full[:, sls[h]] * beta[:, h:h + 1]
        rhs = jnp.concatenate([vh, wkbs[h] * dec_exp], axis=1)
        res = jnp.dot(A_h, rhs, preferred_element_type=f32)
        v2_ref[:, sls[h]] = res[:, :d]
        wkcd_ref[:, sls[h]] = res[:, d:]
        rkdec_ref[:, sls[h]] = rk_all[:, sls[h]] * dec_exp
        dw = jnp.exp(dec_h[C - 1:C, :] - dec_h)
        wkdw_ref[:, sls[h]] = wk_all[:, sls[h]] * dw

    @pl.when(k == 0)
    def _init_S():
        S_ref[...] = jnp.zeros_like(S_ref)


def kernel(x, W_write, W_gate, W_out, W_beta, W_alpha, dt_bias, A_log):
    B, T, D = x.shape
    H = A_log.shape[0]
    d = D // H
    C = _C
    N = T // C

    wwT = W_write.T.astype(jnp.bfloat16)
    woT = W_out.T.astype(jnp.bfloat16)
    wbgT = jnp.concatenate([W_beta.T, W_gate.T], axis=1)
    waT = W_alpha.T
    dtb = dt_bias.reshape(1, H).astype(jnp.float32)
    nega = (-jnp.exp(A_log)).reshape(1, H).astype(jnp.float32)

    const = lambda b, k: (0, 0)
    out = pl.pallas_call(
        _block_kernel,
        grid=(B, N + 1),
        in_specs=[
            pl.BlockSpec((1, C, D), lambda b, k: (b, jnp.maximum(k - 1, 0), 0)),
            pl.BlockSpec((1, C, D), lambda b, k: (b, jnp.minimum(k, N - 1), 0)),
            pl.BlockSpec((D, D), const),
            pl.BlockSpec((D, 2 * H), const),
            pl.BlockSpec((D, H), const),
            pl.BlockSpec((D, D), const),
            pl.BlockSpec((1, H), const),
            pl.BlockSpec((1, H), const),
        ],
        out_specs=pl.BlockSpec((1, C, D), lambda b, k: (b, jnp.maximum(k - 1, 0), 0)),
        out_shape=jax.ShapeDtypeStruct((B, T, D), jnp.float32),
        scratch_shapes=[
            pltpu.VMEM((H, d, d), jnp.float32),
            pltpu.VMEM((1, D), jnp.float32),
            pltpu.VMEM((C, D), jnp.float32),
            pltpu.VMEM((C, D), jnp.float32),
            pltpu.VMEM((C, D), jnp.float32),
            pltpu.VMEM((C, D), jnp.float32),
            pltpu.VMEM((H * C, C), jnp.float32),
            pltpu.VMEM((C, H), jnp.float32),
            pltpu.VMEM((1, H), jnp.float32),
        ],
        compiler_params=pltpu.CompilerParams(
            dimension_semantics=("parallel", "arbitrary"),
        ),
    )(x.astype(jnp.float32), x.astype(jnp.float32), wwT, wbgT, waT, woT,
      dtb, nega)
    return out.astype(x.dtype)


# speedup vs baseline: 1.0011x; 1.0011x over previous
import jax
import jax.numpy as jnp
from jax.experimental import pallas as pl
from jax.experimental.pallas import tpu as pltpu

_C = 64


def _softplus(z):
    return jnp.maximum(z, 0.0) + jnp.log1p(jnp.exp(-jnp.abs(z)))


def _sigmoid(z):
    return 1.0 / (1.0 + jnp.exp(-z))


def _block_kernel(xp_ref, xn_ref, wwT_ref, wbgT_ref, waT_ref, woT_ref,
                  dtb_ref, nega_ref, out_ref, S_ref, prev_ref, v2_ref,
                  wkcd_ref, rkdec_ref, wkdw_ref, attn_ref, gate_ref, sdec_ref):
    C = _C
    H, d, _ = S_ref.shape
    D = H * d
    f32 = jnp.float32
    k = pl.program_id(1)

    sls = [slice(h * d, (h + 1) * d) for h in range(H)]

    boths = []
    for h in range(H):
        lhs = jnp.concatenate([wkcd_ref[:, sls[h]], rkdec_ref[:, sls[h]]],
                              axis=0)
        boths.append(jnp.dot(lhs, S_ref[h], preferred_element_type=f32))

    xb = xn_ref[0]
    bg = jnp.dot(xb, wbgT_ref[...], preferred_element_type=f32)
    beta = _sigmoid(bg[:, 0:H])
    gate = _sigmoid(bg[:, H:2 * H])
    z = jnp.dot(xb, waT_ref[...], preferred_element_type=f32) + dtb_ref[...]
    decay = nega_ref[...] * _softplus(z)
    v_full = jnp.dot(xb, wwT_ref[...], preferred_element_type=f32)

    vns, o_parts = [], []
    for h in range(H):
        v_new = v2_ref[:, sls[h]] - boths[h][:C]
        vns.append(v_new)
        o_parts.append(boths[h][C:] + jnp.dot(
            attn_ref[h * C:(h + 1) * C, :], v_new, preferred_element_type=f32))

    ri = jax.lax.broadcasted_iota(jnp.int32, (C, C), 0)
    ci = jax.lax.broadcasted_iota(jnp.int32, (C, C), 1)
    eyeC = jnp.where(ri == ci, f32(1.0), f32(0.0))
    L1 = jnp.where(ri >= ci, f32(1.0), f32(0.0))
    subD = jnp.where(ri == ci + 1, f32(1.0), f32(0.0))
    m0 = jax.lax.broadcasted_iota(jnp.int32, (C, D), 0) == 0
    G = min(8, H)
    riS = jax.lax.broadcasted_iota(jnp.int32, (C, G * C), 0)
    ciS = jax.lax.broadcasted_iota(jnp.int32, (C, G * C), 1)
    eyeS = jnp.where((ciS & (C - 1)) == riS, f32(1.0), f32(0.0))
    rB = jax.lax.broadcasted_iota(jnp.int32, (G * C, G * C), 0)
    cB = jax.lax.broadcasted_iota(jnp.int32, (G * C, G * C), 1)
    blk = (rB >> 6) == (cB >> 6)

    dec = jnp.dot(L1, decay, preferred_element_type=f32)
    decT = jax.lax.dot_general(dec, eyeC, (((0,), (0,)), ((), ())),
                               preferred_element_type=f32)

    rks = []
    for h in range(H):
        xh = xb[:, sls[h]]
        inv = 1.0 / jnp.maximum(
            jnp.sqrt(jnp.sum(xh * xh, axis=1, keepdims=True)), f32(1e-12))
        rks.append(xh * inv)
    rk_all = jnp.concatenate(rks, axis=1)
    prev_row = jnp.where(k == 0, f32(0.0), prev_ref[...])
    wk_all = jnp.dot(subD, rk_all, preferred_element_type=f32) \
        + jnp.where(m0, jnp.broadcast_to(prev_row, (C, D)), f32(0.0))
    prev_ref[...] = rk_all[C - 1:C, :]

    Ms, wkbs = [], []
    for h in range(H):
        wkb = wk_all[:, sls[h]] * beta[:, h:h + 1]
        rawb = jax.lax.dot_general(
            jnp.concatenate([wkb, rk_all[:, sls[h]]], axis=0), wk_all[:, sls[h]],
            (((1,), (1,)), ((), ())), preferred_element_type=f32)
        L = jnp.exp(jnp.where(ri >= ci, dec[:, h:h + 1] - decT[h:h + 1, :],
                              f32(-1e30)))
        Ms.append(jnp.where(ri > ci, rawb[:C] * L, f32(0.0)))
        attn_ref[h * C:(h + 1) * C, :] = rawb[C:] * L
        wkbs.append(wkb)

    def bd(Q):
        return jnp.where(blk, pltpu.repeat(Q, G, axis=0), f32(0.0))

    A_gs = []
    for g in range(H // G):
        P = -jnp.concatenate(Ms[g * G:(g + 1) * G], axis=1)
        A = eyeS + P
        Q = jnp.dot(P, bd(P), preferred_element_type=f32)
        for i in range(5):
            Qb = bd(Q)
            if i < 4:
                both2 = jnp.dot(jnp.concatenate([A, Q], axis=0), Qb,
                                preferred_element_type=f32)
                A = A + both2[:C]
                Q = both2[C:]
            else:
                A = A + jnp.dot(A, Qb, preferred_element_type=f32)
        A_gs.append(A)
    A = jnp.concatenate(A_gs, axis=1)

    S_news, outs = [], []
    for h in range(H):
        S_news.append(S_ref[h] * sdec_ref[:, h:h + 1] + jax.lax.dot_general(
            wkdw_ref[:, sls[h]], vns[h], (((0,), (0,)), ((), ())),
            preferred_element_type=f32))
        outs.append(o_parts[h] * gate_ref[:, h:h + 1])
    for h in range(H):
        S_ref[h] = S_news[h]
    o_full = jnp.concatenate(outs, axis=1)
    out_ref[0] = xp_ref[0] + jnp.dot(o_full, woT_ref[...],
                                     preferred_element_type=f32)

    sdec_ref[...] = jnp.exp(dec[C - 1:C, :])
    gate_ref[...] = gate
    for h in range(H):
        A_h = A[:, h * C:(h + 1) * C]
        dec_h = dec[:, h:h + 1]
        dec_exp = jnp.exp(dec_h)
        vh = v_full[:, sls[h]] * beta[:, h:h + 1]
        rhs = jnp.concatenate([vh, wkbs[h] * dec_exp], axis=1)
        res = jnp.dot(A_h, rhs, preferred_element_type=f32)
        v2_ref[:, sls[h]] = res[:, :d]
        wkcd_ref[:, sls[h]] = res[:, d:]
        rkdec_ref[:, sls[h]] = rk_all[:, sls[h]] * dec_exp
        dw = jnp.exp(dec_h[C - 1:C, :] - dec_h)
        wkdw_ref[:, sls[h]] = wk_all[:, sls[h]] * dw

    @pl.when(k == 0)
    def _init_S():
        S_ref[...] = jnp.zeros_like(S_ref)


def kernel(x, W_write, W_gate, W_out, W_beta, W_alpha, dt_bias, A_log):
    B, T, D = x.shape
    H = A_log.shape[0]
    d = D // H
    C = _C
    N = T // C

    wwT = W_write.T
    woT = W_out.T
    wbgT = jnp.concatenate([W_beta.T, W_gate.T], axis=1)
    waT = W_alpha.T
    dtb = dt_bias.reshape(1, H).astype(jnp.float32)
    nega = (-jnp.exp(A_log)).reshape(1, H).astype(jnp.float32)

    const = lambda b, k: (0, 0)
    out = pl.pallas_call(
        _block_kernel,
        grid=(B, N + 1),
        in_specs=[
            pl.BlockSpec((1, C, D), lambda b, k: (b, jnp.maximum(k - 1, 0), 0)),
            pl.BlockSpec((1, C, D), lambda b, k: (b, jnp.minimum(k, N - 1), 0)),
            pl.BlockSpec((D, D), const),
            pl.BlockSpec((D, 2 * H), const),
            pl.BlockSpec((D, H), const),
            pl.BlockSpec((D, D), const),
            pl.BlockSpec((1, H), const),
            pl.BlockSpec((1, H), const),
        ],
        out_specs=pl.BlockSpec((1, C, D), lambda b, k: (b, jnp.maximum(k - 1, 0), 0)),
        out_shape=jax.ShapeDtypeStruct((B, T, D), jnp.float32),
        scratch_shapes=[
            pltpu.VMEM((H, d, d), jnp.float32),
            pltpu.VMEM((1, D), jnp.float32),
            pltpu.VMEM((C, D), jnp.float32),
            pltpu.VMEM((C, D), jnp.float32),
            pltpu.VMEM((C, D), jnp.float32),
            pltpu.VMEM((C, D), jnp.float32),
            pltpu.VMEM((H * C, C), jnp.float32),
            pltpu.VMEM((C, H), jnp.float32),
            pltpu.VMEM((1, H), jnp.float32),
        ],
        compiler_params=pltpu.CompilerParams(
            dimension_semantics=("parallel", "arbitrary"),
        ),
    )(x.astype(jnp.float32), x.astype(jnp.float32), wwT, wbgT, waT, woT,
      dtb, nega)
    return out.astype(x.dtype)


# speedup vs baseline: 1.2561x; 1.2547x over previous
import jax
import jax.numpy as jnp
from jax.experimental import pallas as pl
from jax.experimental.pallas import tpu as pltpu

_C = 64


def _softplus(z):
    return jnp.maximum(z, 0.0) + jnp.log1p(jnp.exp(-jnp.abs(z)))


def _sigmoid(z):
    return 1.0 / (1.0 + jnp.exp(-z))


def _block_kernel(xp_ref, xn_ref, wwT_ref, wbgT_ref, waT_ref, woT_ref,
                  dtb_ref, nega_ref, out_ref, S_ref, prev_ref, v2_ref,
                  wkcd_ref, rkdec_ref, wkdw_ref, attn_ref, gate_ref, sdec_ref):
    C = _C
    H, d, _ = S_ref.shape
    D = H * d
    C2 = 2 * C
    f32 = jnp.float32
    k = pl.program_id(1)

    sls = [slice(h * d, (h + 1) * d) for h in range(H)]
    rsl = [slice(0, C), slice(C, C2)]

    o_chunks = []
    for c in range(2):
        boths = []
        for h in range(H):
            lhs = jnp.concatenate([wkcd_ref[rsl[c], sls[h]],
                                   rkdec_ref[rsl[c], sls[h]]], axis=0)
            boths.append(jnp.dot(lhs, S_ref[h], preferred_element_type=f32))
        if c == 0:
            xb = xn_ref[0]
            bg = jnp.dot(xb, wbgT_ref[...], preferred_element_type=f32)
            beta = _sigmoid(bg[:, 0:H])
            gate = _sigmoid(bg[:, H:2 * H])
            z = jnp.dot(xb, waT_ref[...], preferred_element_type=f32) \
                + dtb_ref[...]
            decay = nega_ref[...] * _softplus(z)
            v_full = jnp.dot(xb, wwT_ref[...], preferred_element_type=f32)
        vns, o_parts = [], []
        for h in range(H):
            v_new = v2_ref[rsl[c], sls[h]] - boths[h][:C]
            vns.append(v_new)
            o_parts.append(boths[h][C:] + jnp.dot(
                attn_ref[(c * H + h) * C:(c * H + h + 1) * C, :], v_new,
                preferred_element_type=f32))
        S_news, outs = [], []
        for h in range(H):
            S_news.append(
                S_ref[h] * sdec_ref[c:c + 1, h:h + 1] + jax.lax.dot_general(
                    wkdw_ref[rsl[c], sls[h]], vns[h], (((0,), (0,)), ((), ())),
                    preferred_element_type=f32))
            outs.append(o_parts[h] * gate_ref[rsl[c], h:h + 1])
        for h in range(H):
            S_ref[h] = S_news[h]
        o_chunks.append(jnp.concatenate(outs, axis=1))

    o_full = jnp.concatenate(o_chunks, axis=0)
    out_ref[0] = xp_ref[0] + jnp.dot(o_full, woT_ref[...],
                                     preferred_element_type=f32)

    ri = jax.lax.broadcasted_iota(jnp.int32, (C, C), 0)
    ci = jax.lax.broadcasted_iota(jnp.int32, (C, C), 1)
    r2 = jax.lax.broadcasted_iota(jnp.int32, (C2, C2), 0)
    c2 = jax.lax.broadcasted_iota(jnp.int32, (C2, C2), 1)
    eyeC2 = jnp.where(r2 == c2, f32(1.0), f32(0.0))
    samehalf = (r2 >> 6) == (c2 >> 6)
    L1 = jnp.where((r2 >= c2) & samehalf, f32(1.0), f32(0.0))
    subD = jnp.where(r2 == c2 + 1, f32(1.0), f32(0.0))
    m0 = jax.lax.broadcasted_iota(jnp.int32, (C2, D), 0) == 0
    riS = jax.lax.broadcasted_iota(jnp.int32, (C, H * C), 0)
    ciS = jax.lax.broadcasted_iota(jnp.int32, (C, H * C), 1)
    eyeS = jnp.where((ciS & (C - 1)) == riS, f32(1.0), f32(0.0))
    rB = jax.lax.broadcasted_iota(jnp.int32, (H * C, H * C), 0)
    cB = jax.lax.broadcasted_iota(jnp.int32, (H * C, H * C), 1)
    blk = (rB >> 6) == (cB >> 6)

    dec = jnp.dot(L1, decay, preferred_element_type=f32)
    decT = jax.lax.dot_general(dec, eyeC2, (((0,), (0,)), ((), ())),
                               preferred_element_type=f32)

    rks = []
    for h in range(H):
        xh = xb[:, sls[h]]
        inv = 1.0 / jnp.maximum(
            jnp.sqrt(jnp.sum(xh * xh, axis=1, keepdims=True)), f32(1e-12))
        rks.append(xh * inv)
    rk_all = jnp.concatenate(rks, axis=1)
    prev_row = jnp.where(k == 0, f32(0.0), prev_ref[...])
    wk_all = jnp.dot(subD, rk_all, preferred_element_type=f32) \
        + jnp.where(m0, jnp.broadcast_to(prev_row, (C2, D)), f32(0.0))
    prev_ref[...] = rk_all[C2 - 1:C2, :]

    Ms, wkbs = [[], []], [[], []]
    for c in range(2):
        for h in range(H):
            wkb = wk_all[rsl[c], sls[h]] * beta[rsl[c], h:h + 1]
            rawb = jax.lax.dot_general(
                jnp.concatenate([wkb, rk_all[rsl[c], sls[h]]], axis=0),
                wk_all[rsl[c], sls[h]],
                (((1,), (1,)), ((), ())), preferred_element_type=f32)
            L = jnp.exp(jnp.where(
                ri >= ci,
                dec[rsl[c], h:h + 1] - decT[h:h + 1, c * C:(c + 1) * C],
                f32(-1e30)))
            Ms[c].append(jnp.where(ri > ci, rawb[:C] * L, f32(0.0)))
            attn_ref[(c * H + h) * C:(c * H + h + 1) * C, :] = rawb[C:] * L
            wkbs[c].append(wkb)

    def bd(Q):
        return jnp.where(blk, pltpu.repeat(Q, H, axis=0), f32(0.0))

    A_cs = []
    for c in range(2):
        P = -jnp.concatenate(Ms[c], axis=1)
        A = eyeS + P
        Q = jnp.dot(P, bd(P), preferred_element_type=f32)
        for i in range(5):
            Qb = bd(Q)
            if i < 4:
                both2 = jnp.dot(jnp.concatenate([A, Q], axis=0), Qb,
                                preferred_element_type=f32)
                A = A + both2[:C]
                Q = both2[C:]
            else:
                A = A + jnp.dot(A, Qb, preferred_element_type=f32)
        A_cs.append(A)

    sdec_ref[...] = jnp.exp(jnp.concatenate(
        [dec[C - 1:C, :], dec[C2 - 1:C2, :]], axis=0))
    gate_ref[...] = gate
    for c in range(2):
        for h in range(H):
            A_h = A_cs[c][:, h * C:(h + 1) * C]
            dec_h = dec[rsl[c], h:h + 1]
            dec_exp = jnp.exp(dec_h)
            vh = v_full[rsl[c], sls[h]] * beta[rsl[c], h:h + 1]
            rhs = jnp.concatenate([vh, wkbs[c][h] * dec_exp], axis=1)
            res = jnp.dot(A_h, rhs, preferred_element_type=f32)
            v2_ref[rsl[c], sls[h]] = res[:, :d]
            wkcd_ref[rsl[c], sls[h]] = res[:, d:]
            rkdec_ref[rsl[c], sls[h]] = rk_all[rsl[c], sls[h]] * dec_exp
            dw = jnp.exp(dec_h[C - 1:C, :] - dec_h)
            wkdw_ref[rsl[c], sls[h]] = wk_all[rsl[c], sls[h]] * dw

    @pl.when(k == 0)
    def _init_S():
        S_ref[...] = jnp.zeros_like(S_ref)


def kernel(x, W_write, W_gate, W_out, W_beta, W_alpha, dt_bias, A_log):
    B, T, D = x.shape
    H = A_log.shape[0]
    d = D // H
    C = _C
    C2 = 2 * C
    N2 = T // C2

    wwT = W_write.T
    woT = W_out.T
    wbgT = jnp.concatenate([W_beta.T, W_gate.T], axis=1)
    waT = W_alpha.T
    dtb = dt_bias.reshape(1, H).astype(jnp.float32)
    nega = (-jnp.exp(A_log)).reshape(1, H).astype(jnp.float32)

    const = lambda b, k: (0, 0)
    out = pl.pallas_call(
        _block_kernel,
        grid=(B, N2 + 1),
        in_specs=[
            pl.BlockSpec((1, C2, D), lambda b, k: (b, jnp.maximum(k - 1, 0), 0)),
            pl.BlockSpec((1, C2, D), lambda b, k: (b, jnp.minimum(k, N2 - 1), 0)),
            pl.BlockSpec((D, D), const),
            pl.BlockSpec((D, 2 * H), const),
            pl.BlockSpec((D, H), const),
            pl.BlockSpec((D, D), const),
            pl.BlockSpec((1, H), const),
            pl.BlockSpec((1, H), const),
        ],
        out_specs=pl.BlockSpec((1, C2, D),
                               lambda b, k: (b, jnp.maximum(k - 1, 0), 0)),
        out_shape=jax.ShapeDtypeStruct((B, T, D), jnp.float32),
        scratch_shapes=[
            pltpu.VMEM((H, d, d), jnp.float32),
            pltpu.VMEM((1, D), jnp.float32),
            pltpu.VMEM((C2, D), jnp.float32),
            pltpu.VMEM((C2, D), jnp.float32),
            pltpu.VMEM((C2, D), jnp.float32),
            pltpu.VMEM((C2, D), jnp.float32),
            pltpu.VMEM((2 * H * C, C), jnp.float32),
            pltpu.VMEM((C2, H), jnp.float32),
            pltpu.VMEM((2, H), jnp.float32),
        ],
        compiler_params=pltpu.CompilerParams(
            dimension_semantics=("parallel", "arbitrary"),
        ),
    )(x.astype(jnp.float32), x.astype(jnp.float32), wwT, wbgT, waT, woT,
      dtb, nega)
    return out.astype(x.dtype)


# speedup vs baseline: 1.3192x; 1.0503x over previous
import jax
import jax.numpy as jnp
from jax.experimental import pallas as pl
from jax.experimental.pallas import tpu as pltpu

_C = 64
_NC = 4


def _softplus(z):
    return jnp.maximum(z, 0.0) + jnp.log1p(jnp.exp(-jnp.abs(z)))


def _sigmoid(z):
    return 1.0 / (1.0 + jnp.exp(-z))


def _block_kernel(xp_ref, xn_ref, wwT_ref, wbgT_ref, waT_ref, woT_ref,
                  dtb_ref, nega_ref, out_ref, S_ref, prev_ref, v2_ref,
                  wkcd_ref, rkdec_ref, wkdw_ref, attn_ref, gate_ref, sdec_ref):
    C = _C
    H, d, _ = S_ref.shape
    D = H * d
    C2 = _NC * C
    f32 = jnp.float32
    k = pl.program_id(1)

    sls = [slice(h * d, (h + 1) * d) for h in range(H)]
    rsl = [slice(c * C, (c + 1) * C) for c in range(_NC)]

    o_chunks = []
    for c in range(_NC):
        boths = []
        for h in range(H):
            lhs = jnp.concatenate([wkcd_ref[rsl[c], sls[h]],
                                   rkdec_ref[rsl[c], sls[h]]], axis=0)
            boths.append(jnp.dot(lhs, S_ref[h], preferred_element_type=f32))
        if c == 0:
            xb = xn_ref[0]
            bg = jnp.dot(xb, wbgT_ref[...], preferred_element_type=f32)
            beta = _sigmoid(bg[:, 0:H])
            gate = _sigmoid(bg[:, H:2 * H])
            z = jnp.dot(xb, waT_ref[...], preferred_element_type=f32) \
                + dtb_ref[...]
            decay = nega_ref[...] * _softplus(z)
            v_full = jnp.dot(xb, wwT_ref[...], preferred_element_type=f32)
        vns, o_parts = [], []
        for h in range(H):
            v_new = v2_ref[rsl[c], sls[h]] - boths[h][:C]
            vns.append(v_new)
            o_parts.append(boths[h][C:] + jnp.dot(
                attn_ref[(c * H + h) * C:(c * H + h + 1) * C, :], v_new,
                preferred_element_type=f32))
        S_news, outs = [], []
        for h in range(H):
            S_news.append(
                S_ref[h] * sdec_ref[c:c + 1, h:h + 1] + jax.lax.dot_general(
                    wkdw_ref[rsl[c], sls[h]], vns[h], (((0,), (0,)), ((), ())),
                    preferred_element_type=f32))
            outs.append(o_parts[h] * gate_ref[rsl[c], h:h + 1])
        for h in range(H):
            S_ref[h] = S_news[h]
        o_chunks.append(jnp.concatenate(outs, axis=1))

    o_full = jnp.concatenate(o_chunks, axis=0)
    out_ref[0] = xp_ref[0] + jnp.dot(o_full, woT_ref[...],
                                     preferred_element_type=f32)

    ri = jax.lax.broadcasted_iota(jnp.int32, (C, C), 0)
    ci = jax.lax.broadcasted_iota(jnp.int32, (C, C), 1)
    r2 = jax.lax.broadcasted_iota(jnp.int32, (C2, C2), 0)
    c2 = jax.lax.broadcasted_iota(jnp.int32, (C2, C2), 1)
    eyeC2 = jnp.where(r2 == c2, f32(1.0), f32(0.0))
    samehalf = (r2 >> 6) == (c2 >> 6)
    L1 = jnp.where((r2 >= c2) & samehalf, f32(1.0), f32(0.0))
    subD = jnp.where(r2 == c2 + 1, f32(1.0), f32(0.0))
    m0 = jax.lax.broadcasted_iota(jnp.int32, (C2, D), 0) == 0
    riS = jax.lax.broadcasted_iota(jnp.int32, (C, H * C), 0)
    ciS = jax.lax.broadcasted_iota(jnp.int32, (C, H * C), 1)
    eyeS = jnp.where((ciS & (C - 1)) == riS, f32(1.0), f32(0.0))
    rB = jax.lax.broadcasted_iota(jnp.int32, (H * C, H * C), 0)
    cB = jax.lax.broadcasted_iota(jnp.int32, (H * C, H * C), 1)
    blk = (rB >> 6) == (cB >> 6)

    dec = jnp.dot(L1, decay, preferred_element_type=f32)
    decT = jax.lax.dot_general(dec, eyeC2, (((0,), (0,)), ((), ())),
                               preferred_element_type=f32)

    rks = []
    for h in range(H):
        xh = xb[:, sls[h]]
        inv = 1.0 / jnp.maximum(
            jnp.sqrt(jnp.sum(xh * xh, axis=1, keepdims=True)), f32(1e-12))
        rks.append(xh * inv)
    rk_all = jnp.concatenate(rks, axis=1)
    prev_row = jnp.where(k == 0, f32(0.0), prev_ref[...])
    wk_all = jnp.dot(subD, rk_all, preferred_element_type=f32) \
        + jnp.where(m0, jnp.broadcast_to(prev_row, (C2, D)), f32(0.0))
    prev_ref[...] = rk_all[C2 - 1:C2, :]

    Ms = [[] for _ in range(_NC)]
    wkbs = [[] for _ in range(_NC)]
    for c in range(_NC):
        for h in range(H):
            wkb = wk_all[rsl[c], sls[h]] * beta[rsl[c], h:h + 1]
            rawb = jax.lax.dot_general(
                jnp.concatenate([wkb, rk_all[rsl[c], sls[h]]], axis=0),
                wk_all[rsl[c], sls[h]],
                (((1,), (1,)), ((), ())), preferred_element_type=f32)
            L = jnp.exp(jnp.where(
                ri >= ci,
                dec[rsl[c], h:h + 1] - decT[h:h + 1, c * C:(c + 1) * C],
                f32(-1e30)))
            Ms[c].append(jnp.where(ri > ci, rawb[:C] * L, f32(0.0)))
            attn_ref[(c * H + h) * C:(c * H + h + 1) * C, :] = rawb[C:] * L
            wkbs[c].append(wkb)

    def bd(Q):
        return jnp.where(blk, pltpu.repeat(Q, H, axis=0), f32(0.0))

    A_cs = []
    for c in range(_NC):
        P = -jnp.concatenate(Ms[c], axis=1)
        A = eyeS + P
        Q = jnp.dot(P, bd(P), preferred_element_type=f32)
        for i in range(5):
            Qb = bd(Q)
            if i < 4:
                both2 = jnp.dot(jnp.concatenate([A, Q], axis=0), Qb,
                                preferred_element_type=f32)
                A = A + both2[:C]
                Q = both2[C:]
            else:
                A = A + jnp.dot(A, Qb, preferred_element_type=f32)
        A_cs.append(A)

    sdec_ref[...] = jnp.exp(jnp.concatenate(
        [dec[(cc + 1) * C - 1:(cc + 1) * C, :] for cc in range(_NC)],
        axis=0))
    gate_ref[...] = gate
    for c in range(_NC):
        for h in range(H):
            A_h = A_cs[c][:, h * C:(h + 1) * C]
            dec_h = dec[rsl[c], h:h + 1]
            dec_exp = jnp.exp(dec_h)
            vh = v_full[rsl[c], sls[h]] * beta[rsl[c], h:h + 1]
            rhs = jnp.concatenate([vh, wkbs[c][h] * dec_exp], axis=1)
            res = jnp.dot(A_h, rhs, preferred_element_type=f32)
            v2_ref[rsl[c], sls[h]] = res[:, :d]
            wkcd_ref[rsl[c], sls[h]] = res[:, d:]
            rkdec_ref[rsl[c], sls[h]] = rk_all[rsl[c], sls[h]] * dec_exp
            dw = jnp.exp(dec_h[C - 1:C, :] - dec_h)
            wkdw_ref[rsl[c], sls[h]] = wk_all[rsl[c], sls[h]] * dw

    @pl.when(k == 0)
    def _init_S():
        S_ref[...] = jnp.zeros_like(S_ref)


def kernel(x, W_write, W_gate, W_out, W_beta, W_alpha, dt_bias, A_log):
    B, T, D = x.shape
    H = A_log.shape[0]
    d = D // H
    C = _C
    C2 = _NC * C
    N2 = T // C2

    wwT = W_write.T
    woT = W_out.T
    wbgT = jnp.concatenate([W_beta.T, W_gate.T], axis=1)
    waT = W_alpha.T
    dtb = dt_bias.reshape(1, H).astype(jnp.float32)
    nega = (-jnp.exp(A_log)).reshape(1, H).astype(jnp.float32)

    const = lambda b, k: (0, 0)
    out = pl.pallas_call(
        _block_kernel,
        grid=(B, N2 + 1),
        in_specs=[
            pl.BlockSpec((1, C2, D), lambda b, k: (b, jnp.maximum(k - 1, 0), 0)),
            pl.BlockSpec((1, C2, D), lambda b, k: (b, jnp.minimum(k, N2 - 1), 0)),
            pl.BlockSpec((D, D), const),
            pl.BlockSpec((D, 2 * H), const),
            pl.BlockSpec((D, H), const),
            pl.BlockSpec((D, D), const),
            pl.BlockSpec((1, H), const),
            pl.BlockSpec((1, H), const),
        ],
        out_specs=pl.BlockSpec((1, C2, D),
                               lambda b, k: (b, jnp.maximum(k - 1, 0), 0)),
        out_shape=jax.ShapeDtypeStruct((B, T, D), jnp.float32),
        scratch_shapes=[
            pltpu.VMEM((H, d, d), jnp.float32),
            pltpu.VMEM((1, D), jnp.float32),
            pltpu.VMEM((C2, D), jnp.float32),
            pltpu.VMEM((C2, D), jnp.float32),
            pltpu.VMEM((C2, D), jnp.float32),
            pltpu.VMEM((C2, D), jnp.float32),
            pltpu.VMEM((_NC * H * C, C), jnp.float32),
            pltpu.VMEM((C2, H), jnp.float32),
            pltpu.VMEM((_NC, H), jnp.float32),
        ],
        compiler_params=pltpu.CompilerParams(
            dimension_semantics=("parallel", "arbitrary"),
        ),
    )(x.astype(jnp.float32), x.astype(jnp.float32), wwT, wbgT, waT, woT,
      dtb, nega)
    return out.astype(x.dtype)


# speedup vs baseline: 1.3287x; 1.0072x over previous
import jax
import jax.numpy as jnp
from jax.experimental import pallas as pl
from jax.experimental.pallas import tpu as pltpu

_C = 64
_NC = 8


def _softplus(z):
    return jnp.maximum(z, 0.0) + jnp.log1p(jnp.exp(-jnp.abs(z)))


def _sigmoid(z):
    return 1.0 / (1.0 + jnp.exp(-z))


def _block_kernel(xp_ref, xn_ref, wwT_ref, wbgT_ref, waT_ref, woT_ref,
                  dtb_ref, nega_ref, out_ref, S_ref, prev_ref, v2_ref,
                  wkcd_ref, rkdec_ref, wkdw_ref, attn_ref, gate_ref, sdec_ref):
    C = _C
    H, d, _ = S_ref.shape
    D = H * d
    C2 = _NC * C
    f32 = jnp.float32
    k = pl.program_id(1)

    sls = [slice(h * d, (h + 1) * d) for h in range(H)]
    rsl = [slice(c * C, (c + 1) * C) for c in range(_NC)]

    o_chunks = []
    for c in range(_NC):
        boths = []
        for h in range(H):
            lhs = jnp.concatenate([wkcd_ref[rsl[c], sls[h]],
                                   rkdec_ref[rsl[c], sls[h]]], axis=0)
            boths.append(jnp.dot(lhs, S_ref[h], preferred_element_type=f32))
        if c == 0:
            xb = xn_ref[0]
            bg = jnp.dot(xb, wbgT_ref[...], preferred_element_type=f32)
            beta = _sigmoid(bg[:, 0:H])
            gate = _sigmoid(bg[:, H:2 * H])
            z = jnp.dot(xb, waT_ref[...], preferred_element_type=f32) \
                + dtb_ref[...]
            decay = nega_ref[...] * _softplus(z)
            v_full = jnp.dot(xb, wwT_ref[...], preferred_element_type=f32)
        vns, o_parts = [], []
        for h in range(H):
            v_new = v2_ref[rsl[c], sls[h]] - boths[h][:C]
            vns.append(v_new)
            o_parts.append(boths[h][C:] + jnp.dot(
                attn_ref[(c * H + h) * C:(c * H + h + 1) * C, :], v_new,
                preferred_element_type=f32))
        S_news, outs = [], []
        for h in range(H):
            S_news.append(
                S_ref[h] * sdec_ref[c:c + 1, h:h + 1] + jax.lax.dot_general(
                    wkdw_ref[rsl[c], sls[h]], vns[h], (((0,), (0,)), ((), ())),
                    preferred_element_type=f32))
            outs.append(o_parts[h] * gate_ref[rsl[c], h:h + 1])
        for h in range(H):
            S_ref[h] = S_news[h]
        o_chunks.append(jnp.concatenate(outs, axis=1))

    o_full = jnp.concatenate(o_chunks, axis=0)
    out_ref[0] = xp_ref[0] + jnp.dot(o_full, woT_ref[...],
                                     preferred_element_type=f32)

    ri = jax.lax.broadcasted_iota(jnp.int32, (C, C), 0)
    ci = jax.lax.broadcasted_iota(jnp.int32, (C, C), 1)
    r2 = jax.lax.broadcasted_iota(jnp.int32, (C2, C2), 0)
    c2 = jax.lax.broadcasted_iota(jnp.int32, (C2, C2), 1)
    eyeC2 = jnp.where(r2 == c2, f32(1.0), f32(0.0))
    samehalf = (r2 >> 6) == (c2 >> 6)
    L1 = jnp.where((r2 >= c2) & samehalf, f32(1.0), f32(0.0))
    subD = jnp.where(r2 == c2 + 1, f32(1.0), f32(0.0))
    m0 = jax.lax.broadcasted_iota(jnp.int32, (C2, D), 0) == 0
    riS = jax.lax.broadcasted_iota(jnp.int32, (C, H * C), 0)
    ciS = jax.lax.broadcasted_iota(jnp.int32, (C, H * C), 1)
    eyeS = jnp.where((ciS & (C - 1)) == riS, f32(1.0), f32(0.0))
    rB = jax.lax.broadcasted_iota(jnp.int32, (H * C, H * C), 0)
    cB = jax.lax.broadcasted_iota(jnp.int32, (H * C, H * C), 1)
    blk = (rB >> 6) == (cB >> 6)

    dec = jnp.dot(L1, decay, preferred_element_type=f32)
    decT = jax.lax.dot_general(dec, eyeC2, (((0,), (0,)), ((), ())),
                               preferred_element_type=f32)

    rks = []
    for h in range(H):
        xh = xb[:, sls[h]]
        inv = 1.0 / jnp.maximum(
            jnp.sqrt(jnp.sum(xh * xh, axis=1, keepdims=True)), f32(1e-12))
        rks.append(xh * inv)
    rk_all = jnp.concatenate(rks, axis=1)
    prev_row = jnp.where(k == 0, f32(0.0), prev_ref[...])
    wk_all = jnp.dot(subD, rk_all, preferred_element_type=f32) \
        + jnp.where(m0, jnp.broadcast_to(prev_row, (C2, D)), f32(0.0))
    prev_ref[...] = rk_all[C2 - 1:C2, :]

    Ms = [[] for _ in range(_NC)]
    wkbs = [[] for _ in range(_NC)]
    for c in range(_NC):
        for h in range(H):
            wkb = wk_all[rsl[c], sls[h]] * beta[rsl[c], h:h + 1]
            rawb = jax.lax.dot_general(
                jnp.concatenate([wkb, rk_all[rsl[c], sls[h]]], axis=0),
                wk_all[rsl[c], sls[h]],
                (((1,), (1,)), ((), ())), preferred_element_type=f32)
            L = jnp.exp(jnp.where(
                ri >= ci,
                dec[rsl[c], h:h + 1] - decT[h:h + 1, c * C:(c + 1) * C],
                f32(-1e30)))
            Ms[c].append(jnp.where(ri > ci, rawb[:C] * L, f32(0.0)))
            attn_ref[(c * H + h) * C:(c * H + h + 1) * C, :] = rawb[C:] * L
            wkbs[c].append(wkb)

    def bd(Q):
        return jnp.where(blk, pltpu.repeat(Q, H, axis=0), f32(0.0))

    A_cs = []
    for c in range(_NC):
        P = -jnp.concatenate(Ms[c], axis=1)
        A = eyeS + P
        Q = jnp.dot(P, bd(P), preferred_element_type=f32)
        for i in range(5):
            Qb = bd(Q)
            if i < 4:
                both2 = jnp.dot(jnp.concatenate([A, Q], axis=0), Qb,
                                preferred_element_type=f32)
                A = A + both2[:C]
                Q = both2[C:]
            else:
                A = A + jnp.dot(A, Qb, preferred_element_type=f32)
        A_cs.append(A)

    sdec_ref[...] = jnp.exp(jnp.concatenate(
        [dec[(cc + 1) * C - 1:(cc + 1) * C, :] for cc in range(_NC)],
        axis=0))
    gate_ref[...] = gate
    for c in range(_NC):
        for h in range(H):
            A_h = A_cs[c][:, h * C:(h + 1) * C]
            dec_h = dec[rsl[c], h:h + 1]
            dec_exp = jnp.exp(dec_h)
            vh = v_full[rsl[c], sls[h]] * beta[rsl[c], h:h + 1]
            rhs = jnp.concatenate([vh, wkbs[c][h] * dec_exp], axis=1)
            res = jnp.dot(A_h, rhs, preferred_element_type=f32)
            v2_ref[rsl[c], sls[h]] = res[:, :d]
            wkcd_ref[rsl[c], sls[h]] = res[:, d:]
            rkdec_ref[rsl[c], sls[h]] = rk_all[rsl[c], sls[h]] * dec_exp
            dw = jnp.exp(dec_h[C - 1:C, :] - dec_h)
            wkdw_ref[rsl[c], sls[h]] = wk_all[rsl[c], sls[h]] * dw

    @pl.when(k == 0)
    def _init_S():
        S_ref[...] = jnp.zeros_like(S_ref)


def kernel(x, W_write, W_gate, W_out, W_beta, W_alpha, dt_bias, A_log):
    B, T, D = x.shape
    H = A_log.shape[0]
    d = D // H
    C = _C
    C2 = _NC * C
    N2 = T // C2

    wwT = W_write.T
    woT = W_out.T
    wbgT = jnp.concatenate([W_beta.T, W_gate.T], axis=1)
    waT = W_alpha.T
    dtb = dt_bias.reshape(1, H).astype(jnp.float32)
    nega = (-jnp.exp(A_log)).reshape(1, H).astype(jnp.float32)

    const = lambda b, k: (0, 0)
    out = pl.pallas_call(
        _block_kernel,
        grid=(B, N2 + 1),
        in_specs=[
            pl.BlockSpec((1, C2, D), lambda b, k: (b, jnp.maximum(k - 1, 0), 0)),
            pl.BlockSpec((1, C2, D), lambda b, k: (b, jnp.minimum(k, N2 - 1), 0)),
            pl.BlockSpec((D, D), const),
            pl.BlockSpec((D, 2 * H), const),
            pl.BlockSpec((D, H), const),
            pl.BlockSpec((D, D), const),
            pl.BlockSpec((1, H), const),
            pl.BlockSpec((1, H), const),
        ],
        out_specs=pl.BlockSpec((1, C2, D),
                               lambda b, k: (b, jnp.maximum(k - 1, 0), 0)),
        out_shape=jax.ShapeDtypeStruct((B, T, D), jnp.float32),
        scratch_shapes=[
            pltpu.VMEM((H, d, d), jnp.float32),
            pltpu.VMEM((1, D), jnp.float32),
            pltpu.VMEM((C2, D), jnp.float32),
            pltpu.VMEM((C2, D), jnp.float32),
            pltpu.VMEM((C2, D), jnp.float32),
            pltpu.VMEM((C2, D), jnp.float32),
            pltpu.VMEM((_NC * H * C, C), jnp.float32),
            pltpu.VMEM((C2, H), jnp.float32),
            pltpu.VMEM((_NC, H), jnp.float32),
        ],
        compiler_params=pltpu.CompilerParams(
            dimension_semantics=("parallel", "arbitrary"),
        ),
    )(x.astype(jnp.float32), x.astype(jnp.float32), wwT, wbgT, waT, woT,
      dtb, nega)
    return out.astype(x.dtype)
